# baseline (speedup 1.0000x reference)
import functools

import jax
import jax.numpy as jnp
from jax.experimental import pallas as pl
from jax.experimental.pallas import tpu as pltpu


def _conv3x3_kernel(x_hbm, w_ref, o_ref, xin_ref, buf_ref, sem,
                    *, W, L, Mg, N):
    C = x_hbm.shape[1]
    H = x_hbm.shape[2]
    bf16 = jnp.bfloat16
    g = pl.program_id(0)

    @pl.when(g == 0)
    def _():
        pltpu.make_async_copy(x_hbm.at[0], xin_ref.at[0], sem.at[0]).start()

    @pl.when(g + 1 < N)
    def _():
        s = (g + 1) % 2
        pltpu.make_async_copy(x_hbm.at[g + 1], xin_ref.at[s],
                              sem.at[s]).start()

    pltpu.make_async_copy(x_hbm.at[g], xin_ref.at[g % 2],
                          sem.at[g % 2]).wait()

    buf_ref[:, :Mg] = jnp.zeros((C, Mg), bf16)
    buf_ref[:, Mg + L:] = jnp.zeros((C, Mg), bf16)
    buf_ref[:, Mg:Mg + L] = xin_ref[g % 2].reshape(C, L).astype(bf16)

    col = jax.lax.broadcasted_iota(jnp.int32, (1, L), 1) % W
    not_first = (col != 0).astype(jnp.float32)
    not_last = (col != W - 1).astype(jnp.float32)

    def tap(kh, kw):
        off = Mg + (kh - 1) * W + (kw - 1)
        return jnp.dot(w_ref[kh * 3 + kw], buf_ref[:, off:off + L],
                       preferred_element_type=jnp.float32)

    left = tap(0, 0) + tap(1, 0) + tap(2, 0)
    mid = tap(0, 1) + tap(1, 1) + tap(2, 1)
    right = tap(0, 2) + tap(1, 2) + tap(2, 2)
    o_ref[0] = (mid + left * not_first + right * not_last).astype(bf16)


def kernel(x, w):
    N, Cin, H, W = x.shape
    Cout, _, K, _ = w.shape
    assert K == 3
    L = H * W
    Mg = 128

    wt = jnp.transpose(w, (2, 3, 0, 1)).reshape(
        K * K, Cout, Cin).astype(jnp.bfloat16)

    out = pl.pallas_call(
        functools.partial(_conv3x3_kernel, W=W, L=L, Mg=Mg, N=N),
        out_shape=jax.ShapeDtypeStruct((N, Cout, L), jnp.bfloat16),
        grid=(N,),
        in_specs=[
            pl.BlockSpec(memory_space=pl.ANY),
            pl.BlockSpec((K * K, Cout, Cin), lambda n: (0, 0, 0)),
        ],
        out_specs=pl.BlockSpec((1, Cout, L), lambda n: (n, 0, 0)),
        scratch_shapes=[
            pltpu.VMEM((2, Cin, H, W), jnp.float32),
            pltpu.VMEM((Cin, 2 * Mg + L), jnp.bfloat16),
            pltpu.SemaphoreType.DMA((2,)),
        ],
        compiler_params=pltpu.CompilerParams(
            dimension_semantics=("arbitrary",)),
    )(x, wt)
    return out.reshape(N, Cout, H, W).astype(jnp.float32)


# speedup vs baseline: 1.4686x; 1.4686x over previous
import functools

import jax
import jax.numpy as jnp
from jax.experimental import pallas as pl
from jax.experimental.pallas import tpu as pltpu


def _conv3x3_kernel(x_ref, w_ref, o_ref, buf_ref, *, B, W, L, Mg):
    C = x_ref.shape[1]
    bf16 = jnp.bfloat16
    P = L + Mg
    NL = (B - 1) * P + L

    buf_ref[:, :Mg] = jnp.zeros((C, Mg), bf16)
    for b in range(B):
        buf_ref[:, Mg + b * P + L:Mg + (b + 1) * P] = jnp.zeros((C, Mg), bf16)
        buf_ref[:, Mg + b * P:Mg + b * P + L] = x_ref[b].astype(bf16)

    col = jax.lax.broadcasted_iota(jnp.int32, (1, NL), 1) % W
    not_first = (col != 0).astype(jnp.float32)
    not_last = (col != W - 1).astype(jnp.float32)

    def tap(kh, kw):
        off = Mg + (kh - 1) * W + (kw - 1)
        return jnp.dot(w_ref[kh * 3 + kw], buf_ref[:, off:off + NL],
                       preferred_element_type=jnp.float32)

    left = tap(0, 0) + tap(1, 0) + tap(2, 0)
    mid = tap(0, 1) + tap(1, 1) + tap(2, 1)
    right = tap(0, 2) + tap(1, 2) + tap(2, 2)
    res = (mid + left * not_first + right * not_last).astype(bf16)
    for b in range(B):
        o_ref[b] = res[:, b * P:b * P + L]


def kernel(x, w):
    N, Cin, H, W = x.shape
    Cout, _, K, _ = w.shape
    assert K == 3
    L = H * W
    Mg = 128
    B = 4
    assert N % B == 0

    x_flat = x.reshape(N, Cin, L)
    wt = jnp.transpose(w, (2, 3, 0, 1)).reshape(
        K * K, Cout, Cin).astype(jnp.bfloat16)

    out = pl.pallas_call(
        functools.partial(_conv3x3_kernel, B=B, W=W, L=L, Mg=Mg),
        out_shape=jax.ShapeDtypeStruct((N, Cout, L), jnp.bfloat16),
        grid=(N // B,),
        in_specs=[
            pl.BlockSpec((B, Cin, L), lambda n: (n, 0, 0)),
            pl.BlockSpec((K * K, Cout, Cin), lambda n: (0, 0, 0)),
        ],
        out_specs=pl.BlockSpec((B, Cout, L), lambda n: (n, 0, 0)),
        scratch_shapes=[pltpu.VMEM((Cin, Mg + B * (L + Mg)), jnp.bfloat16)],
        compiler_params=pltpu.CompilerParams(
            dimension_semantics=("parallel",)),
    )(x_flat, wt)
    return out.reshape(N, Cout, H, W).astype(jnp.float32)
